# Initial kernel scaffold; baseline (speedup 1.0000x reference)
import jax
import jax.numpy as jnp
from jax.experimental import pallas as pl


def kernel(x, edge_index, W1_l, W1_r, b1, W2_l, W2_r, b2):
    raise NotImplementedError("write your pallas kernel here")


# speedup vs baseline: 11.3594x; 11.3594x over previous
import functools

import jax
import jax.numpy as jnp
from jax import lax
from jax.experimental import pallas as pl
from jax.experimental.pallas import tpu as pltpu
from jax.experimental.pallas import tpu_sc as plsc

N_NODES = 10000
N_EDGES = 320000
D_IN = 128
D_HID = 256
D_OUT = 40
D_OUT_PAD = 48

NC = 2
NS = 16
NW = NC * NS

R = 10240
ROWS_PER_TILE = R // NS
EB = 128
EP = 323584
K = EP // (NW * EB)

BN = 1024


def _sc_aggregate(table, src3, dst3, zrows, width, with_deg, zdeg=None, erow=None):
    mesh = plsc.VectorSubcoreMesh(
        core_axis_name="c", subcore_axis_name="s", num_cores=NC, num_subcores=NS
    )
    out_type = [jax.ShapeDtypeStruct((NC * R, width), jnp.float32)]
    scratch = [
        pltpu.VMEM((K, EB), jnp.int32),
        pltpu.VMEM((K, EB), jnp.int32),
        pltpu.VMEM((EB, width), jnp.float32),
        pltpu.VMEM_SHARED((R, width), jnp.float32),
    ]
    if with_deg:
        out_type.append(jax.ShapeDtypeStruct((NC * R, 16), jnp.float32))
        scratch += [
            pltpu.VMEM((EB, 16), jnp.float32),
            pltpu.VMEM_SHARED((R, 16), jnp.float32),
        ]

    def body(*refs):
        if with_deg:
            (table_h, src_h, dst_h, z_h, zd_h, erow_h, agg_o, deg_o,
             src_v, dst_v, rows_v, acc, erow_v, dacc) = refs
        else:
            (table_h, src_h, dst_h, z_h, agg_o,
             src_v, dst_v, rows_v, acc) = refs
        c = lax.axis_index("c")
        s = lax.axis_index("s")
        w = s * NC + c
        r0 = s * ROWS_PER_TILE
        pltpu.sync_copy(z_h.at[pl.ds(r0, ROWS_PER_TILE)],
                        acc.at[pl.ds(r0, ROWS_PER_TILE)])
        pltpu.sync_copy(src_h.at[w], src_v)
        pltpu.sync_copy(dst_h.at[w], dst_v)
        if with_deg:
            pltpu.sync_copy(zd_h.at[pl.ds(r0, ROWS_PER_TILE)],
                            dacc.at[pl.ds(r0, ROWS_PER_TILE)])
            pltpu.sync_copy(erow_h, erow_v)
        plsc.subcore_barrier()

        def step(j, carry):
            pltpu.sync_copy(table_h.at[src_v.at[j]], rows_v)
            pltpu.sync_copy(rows_v, acc.at[dst_v.at[j]], add=True)
            if with_deg:
                pltpu.sync_copy(erow_v, dacc.at[dst_v.at[j]], add=True)
            return carry

        lax.fori_loop(0, K, step, 0)
        plsc.subcore_barrier()
        pltpu.sync_copy(acc.at[pl.ds(r0, ROWS_PER_TILE)],
                        agg_o.at[pl.ds(c * R + r0, ROWS_PER_TILE)])
        if with_deg:
            pltpu.sync_copy(dacc.at[pl.ds(r0, ROWS_PER_TILE)],
                            deg_o.at[pl.ds(c * R + r0, ROWS_PER_TILE)])

    args = [table, src3, dst3, zrows]
    if with_deg:
        args += [zdeg, erow]
    run = pl.kernel(
        body,
        out_type=out_type,
        mesh=mesh,
        scratch_types=scratch,
        compiler_params=pltpu.CompilerParams(use_tc_tiling_on_sc=False),
    )
    return run(*args)


def _tc_layer1(aggp, degp, xp, w1l, w1r, b1, w2l):
    def body(agg_r, deg_r, x_r, wl_r, wr_r, b1_r, w2_r, h_o, p2_o):
        a = agg_r[0] + agg_r[1]
        deg = deg_r[0][:, 0:1] + deg_r[1][:, 0:1]
        mean = a * (1.0 / jnp.maximum(deg, 1.0))
        h = jnp.dot(mean, wl_r[...], preferred_element_type=jnp.float32)
        h += jnp.dot(x_r[...], wr_r[...], preferred_element_type=jnp.float32)
        h = jnp.maximum(h + b1_r[...], 0.0)
        h_o[...] = h
        p2_o[...] = jnp.dot(h, w2_r[...], preferred_element_type=jnp.float32)

    grid = (R // BN,)
    return pl.pallas_call(
        body,
        grid=grid,
        in_specs=[
            pl.BlockSpec((2, BN, D_IN), lambda g: (0, g, 0)),
            pl.BlockSpec((2, BN, 16), lambda g: (0, g, 0)),
            pl.BlockSpec((BN, D_IN), lambda g: (g, 0)),
            pl.BlockSpec((D_IN, D_HID), lambda g: (0, 0)),
            pl.BlockSpec((D_IN, D_HID), lambda g: (0, 0)),
            pl.BlockSpec((1, D_HID), lambda g: (0, 0)),
            pl.BlockSpec((D_HID, D_OUT_PAD), lambda g: (0, 0)),
        ],
        out_specs=[
            pl.BlockSpec((BN, D_HID), lambda g: (g, 0)),
            pl.BlockSpec((BN, D_OUT_PAD), lambda g: (g, 0)),
        ],
        out_shape=[
            jax.ShapeDtypeStruct((R, D_HID), jnp.float32),
            jax.ShapeDtypeStruct((R, D_OUT_PAD), jnp.float32),
        ],
    )(aggp, degp, xp, w1l, w1r, b1, w2l)


def _tc_layer2(agg2p, degp, h, w2r, b2):
    def body(agg_r, deg_r, h_r, wr_r, b2_r, out_o):
        a = agg_r[0] + agg_r[1]
        deg = deg_r[0][:, 0:1] + deg_r[1][:, 0:1]
        z = a * (1.0 / jnp.maximum(deg, 1.0))
        z += jnp.dot(h_r[...], wr_r[...], preferred_element_type=jnp.float32)
        z += b2_r[...]
        col = lax.broadcasted_iota(jnp.int32, (BN, D_OUT_PAD), 1)
        z = jnp.where(col < D_OUT, z, -1e30)
        m = jnp.max(z, axis=-1, keepdims=True)
        e = jnp.exp(z - m)
        lse = jnp.log(jnp.sum(e, axis=-1, keepdims=True))
        out_o[...] = z - m - lse

    grid = (R // BN,)
    return pl.pallas_call(
        body,
        grid=grid,
        in_specs=[
            pl.BlockSpec((2, BN, D_OUT_PAD), lambda g: (0, g, 0)),
            pl.BlockSpec((2, BN, 16), lambda g: (0, g, 0)),
            pl.BlockSpec((BN, D_HID), lambda g: (g, 0)),
            pl.BlockSpec((D_HID, D_OUT_PAD), lambda g: (0, 0)),
            pl.BlockSpec((1, D_OUT_PAD), lambda g: (0, 0)),
        ],
        out_specs=pl.BlockSpec((BN, D_OUT_PAD), lambda g: (g, 0)),
        out_shape=jax.ShapeDtypeStruct((R, D_OUT_PAD), jnp.float32),
    )(agg2p, degp, h, w2r, b2)


def kernel(x, edge_index, W1_l, W1_r, b1, W2_l, W2_r, b2):
    src = edge_index[0].astype(jnp.int32)
    dst = edge_index[1].astype(jnp.int32)
    npad = EP - N_EDGES
    pad_i = jnp.arange(npad, dtype=jnp.int32)
    src3 = jnp.concatenate([src, pad_i % N_NODES]).reshape(NW, K, EB)
    dst3 = jnp.concatenate([dst, N_NODES + pad_i % (R - N_NODES)]).reshape(NW, K, EB)

    xp = jnp.pad(x, ((0, R - N_NODES), (0, 0)))
    w2l = jnp.pad(W2_l, ((0, 0), (0, D_OUT_PAD - D_OUT)))
    w2r = jnp.pad(W2_r, ((0, 0), (0, D_OUT_PAD - D_OUT)))
    b1r = b1.reshape(1, D_HID)
    b2r = jnp.pad(b2, (0, D_OUT_PAD - D_OUT)).reshape(1, D_OUT_PAD)

    zx = jnp.zeros((R, D_IN), jnp.float32)
    zd = jnp.zeros((R, 16), jnp.float32)
    z2 = jnp.zeros((R, D_OUT_PAD), jnp.float32)
    erow = jnp.zeros((EB, 16), jnp.float32).at[:, 0].set(1.0)

    aggp, degp = _sc_aggregate(xp, src3, dst3, zx, D_IN, True, zdeg=zd, erow=erow)
    aggp = aggp.reshape(NC, R, D_IN)
    degp = degp.reshape(NC, R, 16)

    h, p2 = _tc_layer1(aggp, degp, xp, W1_l, W1_r, b1r, w2l)

    (agg2p,) = _sc_aggregate(p2, src3, dst3, z2, D_OUT_PAD, False)
    agg2p = agg2p.reshape(NC, R, D_OUT_PAD)

    out = _tc_layer2(agg2p, degp, h, w2r, b2r)
    return out[:N_NODES, :D_OUT]


# speedup vs baseline: 15.0004x; 1.3205x over previous
import functools

import jax
import jax.numpy as jnp
from jax import lax
from jax.experimental import pallas as pl
from jax.experimental.pallas import tpu as pltpu
from jax.experimental.pallas import tpu_sc as plsc

N_NODES = 10000
N_EDGES = 320000
D_IN = 128
D_HID = 256
D_OUT = 40
D_OUT_PAD = 48

NC = 2
NS = 16
NW = NC * NS

R = 10240
ROWS_PER_TILE = R // NS
EB = 128
NBUF = 4
EP = 327680
K = EP // (NW * EB)

BN = 1024


def _sc_aggregate(tables, src3, dst3, zrows, width, with_deg, zdeg=None, erow=None):
    npass = len(tables)
    mesh = plsc.VectorSubcoreMesh(
        core_axis_name="c", subcore_axis_name="s", num_cores=NC, num_subcores=NS
    )
    out_type = [jax.ShapeDtypeStruct((NC * R, width), jnp.float32)
                for _ in range(npass)]
    scratch = [
        pltpu.VMEM((K, EB), jnp.int32),
        pltpu.VMEM((K, EB), jnp.int32),
        pltpu.VMEM((NBUF, EB, width), jnp.float32),
        pltpu.VMEM_SHARED((R, width), jnp.float32),
    ]
    scratch += [pltpu.SemaphoreType.DMA] * (2 * NBUF)
    if with_deg:
        out_type.append(jax.ShapeDtypeStruct((NC * R, 16), jnp.float32))
        scratch += [
            pltpu.VMEM((EB, 16), jnp.float32),
            pltpu.VMEM_SHARED((R, 16), jnp.float32),
        ]

    def body(*refs):
        table_h = refs[:npass]
        if with_deg:
            (src_h, dst_h, z_h, zd_h, erow_h, *refs2) = refs[npass:]
            agg_o = refs2[:npass]
            (deg_o, src_v, dst_v, rows_v, acc, *rest) = refs2[npass:]
            gsem = rest[:NBUF]
            ssem = rest[NBUF:2 * NBUF]
            erow_v, dacc = rest[2 * NBUF:]
        else:
            (src_h, dst_h, z_h, *refs2) = refs[npass:]
            agg_o = refs2[:npass]
            (src_v, dst_v, rows_v, acc, *rest) = refs2[npass:]
            gsem = rest[:NBUF]
            ssem = rest[NBUF:2 * NBUF]
        c = lax.axis_index("c")
        s = lax.axis_index("s")
        w = s * NC + c
        r0 = s * ROWS_PER_TILE
        pltpu.sync_copy(src_h.at[w], src_v)
        pltpu.sync_copy(dst_h.at[w], dst_v)
        if with_deg:
            pltpu.sync_copy(zd_h.at[pl.ds(r0, ROWS_PER_TILE)],
                            dacc.at[pl.ds(r0, ROWS_PER_TILE)])
            pltpu.sync_copy(erow_h, erow_v)

        for p in range(npass):
            deg_pass = with_deg and p == 0
            pltpu.sync_copy(z_h.at[pl.ds(r0, ROWS_PER_TILE)],
                            acc.at[pl.ds(r0, ROWS_PER_TILE)])
            plsc.subcore_barrier()

            def gather(j, b, p=p):
                return pltpu.async_copy(table_h[p].at[src_v.at[j]],
                                        rows_v.at[b], gsem[b])

            for b in range(NBUF):
                gather(b, b)

            def step(t, carry, p=p, deg_pass=deg_pass, gather=gather):
                j0 = t * NBUF
                scat = []
                for b in range(NBUF):
                    j = j0 + b
                    pltpu.make_async_copy(table_h[p].at[src_v.at[j]],
                                          rows_v.at[b], gsem[b]).wait()
                    scat.append(pltpu.async_copy(
                        rows_v.at[b], acc.at[dst_v.at[j]], ssem[b], add=True))
                    if deg_pass:
                        scat.append(pltpu.async_copy(
                            erow_v, dacc.at[dst_v.at[j]], ssem[b], add=True))
                for b in range(NBUF):
                    j = j0 + NBUF + b

                    @pl.when(j < K)
                    def _():
                        nb = 2 if deg_pass else 1
                        for d in scat[b * nb:(b + 1) * nb]:
                            d.wait()
                        gather(j, b)

                return carry

            lax.fori_loop(0, K // NBUF, step, 0)
            for b in range(NBUF):
                j = K - NBUF + b
                pltpu.make_async_copy(rows_v.at[b], acc.at[dst_v.at[j]],
                                      ssem[b]).wait()
                if deg_pass:
                    pltpu.make_async_copy(erow_v, dacc.at[dst_v.at[j]],
                                          ssem[b]).wait()
            plsc.subcore_barrier()
            pltpu.sync_copy(acc.at[pl.ds(r0, ROWS_PER_TILE)],
                            agg_o[p].at[pl.ds(c * R + r0, ROWS_PER_TILE)])
        if with_deg:
            pltpu.sync_copy(dacc.at[pl.ds(r0, ROWS_PER_TILE)],
                            deg_o.at[pl.ds(c * R + r0, ROWS_PER_TILE)])

    args = list(tables) + [src3, dst3, zrows]
    if with_deg:
        args += [zdeg, erow]
    run = pl.kernel(
        body,
        out_type=out_type,
        mesh=mesh,
        scratch_types=scratch,
        compiler_params=pltpu.CompilerParams(use_tc_tiling_on_sc=False),
    )
    return run(*args)


def _tc_layer1(agg0p, agg1p, degp, xp, w1l, w1r, b1, w2l):
    HW = D_IN // 2

    def body(a0_r, a1_r, deg_r, x_r, wl_r, wr_r, b1_r, w2_r, h_o, p2_o):
        deg = deg_r[0][:, 0:1] + deg_r[1][:, 0:1]
        rdeg = 1.0 / jnp.maximum(deg, 1.0)
        m0 = (a0_r[0] + a0_r[1]) * rdeg
        m1 = (a1_r[0] + a1_r[1]) * rdeg
        h = jnp.dot(m0, wl_r[0:HW], preferred_element_type=jnp.float32)
        h += jnp.dot(m1, wl_r[HW:D_IN], preferred_element_type=jnp.float32)
        h += jnp.dot(x_r[...], wr_r[...], preferred_element_type=jnp.float32)
        h = jnp.maximum(h + b1_r[...], 0.0)
        h_o[...] = h
        p2_o[...] = jnp.dot(h, w2_r[...], preferred_element_type=jnp.float32)

    grid = (R // BN,)
    return pl.pallas_call(
        body,
        grid=grid,
        in_specs=[
            pl.BlockSpec((2, BN, HW), lambda g: (0, g, 0)),
            pl.BlockSpec((2, BN, HW), lambda g: (0, g, 0)),
            pl.BlockSpec((2, BN, 16), lambda g: (0, g, 0)),
            pl.BlockSpec((BN, D_IN), lambda g: (g, 0)),
            pl.BlockSpec((D_IN, D_HID), lambda g: (0, 0)),
            pl.BlockSpec((D_IN, D_HID), lambda g: (0, 0)),
            pl.BlockSpec((1, D_HID), lambda g: (0, 0)),
            pl.BlockSpec((D_HID, D_OUT_PAD), lambda g: (0, 0)),
        ],
        out_specs=[
            pl.BlockSpec((BN, D_HID), lambda g: (g, 0)),
            pl.BlockSpec((BN, D_OUT_PAD), lambda g: (g, 0)),
        ],
        out_shape=[
            jax.ShapeDtypeStruct((R, D_HID), jnp.float32),
            jax.ShapeDtypeStruct((R, D_OUT_PAD), jnp.float32),
        ],
    )(agg0p, agg1p, degp, xp, w1l, w1r, b1, w2l)


def _tc_layer2(agg2p, degp, h, w2r, b2):
    def body(agg_r, deg_r, h_r, wr_r, b2_r, out_o):
        a = agg_r[0] + agg_r[1]
        deg = deg_r[0][:, 0:1] + deg_r[1][:, 0:1]
        z = a * (1.0 / jnp.maximum(deg, 1.0))
        z += jnp.dot(h_r[...], wr_r[...], preferred_element_type=jnp.float32)
        z += b2_r[...]
        col = lax.broadcasted_iota(jnp.int32, (BN, D_OUT_PAD), 1)
        z = jnp.where(col < D_OUT, z, -1e30)
        m = jnp.max(z, axis=-1, keepdims=True)
        e = jnp.exp(z - m)
        lse = jnp.log(jnp.sum(e, axis=-1, keepdims=True))
        out_o[...] = z - m - lse

    grid = (R // BN,)
    return pl.pallas_call(
        body,
        grid=grid,
        in_specs=[
            pl.BlockSpec((2, BN, D_OUT_PAD), lambda g: (0, g, 0)),
            pl.BlockSpec((2, BN, 16), lambda g: (0, g, 0)),
            pl.BlockSpec((BN, D_HID), lambda g: (g, 0)),
            pl.BlockSpec((D_HID, D_OUT_PAD), lambda g: (0, 0)),
            pl.BlockSpec((1, D_OUT_PAD), lambda g: (0, 0)),
        ],
        out_specs=pl.BlockSpec((BN, D_OUT_PAD), lambda g: (g, 0)),
        out_shape=jax.ShapeDtypeStruct((R, D_OUT_PAD), jnp.float32),
    )(agg2p, degp, h, w2r, b2)


def kernel(x, edge_index, W1_l, W1_r, b1, W2_l, W2_r, b2):
    src = edge_index[0].astype(jnp.int32)
    dst = edge_index[1].astype(jnp.int32)
    npad = EP - N_EDGES
    pad_i = jnp.arange(npad, dtype=jnp.int32)
    src3 = jnp.concatenate([src, pad_i % N_NODES]).reshape(NW, K, EB)
    dst3 = jnp.concatenate([dst, N_NODES + pad_i % (R - N_NODES)]).reshape(NW, K, EB)

    xp = jnp.pad(x, ((0, R - N_NODES), (0, 0)))
    w2l = jnp.pad(W2_l, ((0, 0), (0, D_OUT_PAD - D_OUT)))
    w2r = jnp.pad(W2_r, ((0, 0), (0, D_OUT_PAD - D_OUT)))
    b1r = b1.reshape(1, D_HID)
    b2r = jnp.pad(b2, (0, D_OUT_PAD - D_OUT)).reshape(1, D_OUT_PAD)

    HW = D_IN // 2
    zx = jnp.zeros((R, HW), jnp.float32)
    zd = jnp.zeros((R, 16), jnp.float32)
    z2 = jnp.zeros((R, D_OUT_PAD), jnp.float32)
    erow = jnp.zeros((EB, 16), jnp.float32).at[:, 0].set(1.0)

    xh0 = xp[:, :HW]
    xh1 = xp[:, HW:]
    agg0p, agg1p, degp = _sc_aggregate(
        [xh0, xh1], src3, dst3, zx, HW, True, zdeg=zd, erow=erow)
    agg0p = agg0p.reshape(NC, R, HW)
    agg1p = agg1p.reshape(NC, R, HW)
    degp = degp.reshape(NC, R, 16)

    h, p2 = _tc_layer1(agg0p, agg1p, degp, xp, W1_l, W1_r, b1r, w2l)

    (agg2p,) = _sc_aggregate([p2], src3, dst3, z2, D_OUT_PAD, False)
    agg2p = agg2p.reshape(NC, R, D_OUT_PAD)

    out = _tc_layer2(agg2p, degp, h, w2r, b2r)
    return out[:N_NODES, :D_OUT]


# speedup vs baseline: 15.0705x; 1.0047x over previous
import functools

import jax
import jax.numpy as jnp
from jax import lax
from jax.experimental import pallas as pl
from jax.experimental.pallas import tpu as pltpu
from jax.experimental.pallas import tpu_sc as plsc

N_NODES = 10000
N_EDGES = 320000
D_IN = 128
D_HID = 256
D_OUT = 40
D_OUT_PAD = 48

NC = 2
NS = 16
NW = NC * NS

R = 10240
ROWS_PER_TILE = R // NS
EB = 128
EP = 327680
K = EP // (NW * EB)

BN = 1024


def _sc_aggregate(tables, src3, dst3, zrows, width, with_deg, nbuf, zdeg=None, erow=None):
    npass = len(tables)
    NBUF = nbuf
    mesh = plsc.VectorSubcoreMesh(
        core_axis_name="c", subcore_axis_name="s", num_cores=NC, num_subcores=NS
    )
    out_type = [jax.ShapeDtypeStruct((NC * R, width), jnp.float32)
                for _ in range(npass)]
    scratch = [
        pltpu.VMEM((K, EB), jnp.int32),
        pltpu.VMEM((K, EB), jnp.int32),
        pltpu.VMEM((NBUF, EB, width), jnp.float32),
        pltpu.VMEM_SHARED((R, width), jnp.float32),
    ]
    scratch += [pltpu.SemaphoreType.DMA] * (2 * NBUF)
    if with_deg:
        out_type.append(jax.ShapeDtypeStruct((NC * R, 16), jnp.float32))
        scratch += [
            pltpu.VMEM((EB, 16), jnp.float32),
            pltpu.VMEM_SHARED((R, 16), jnp.float32),
        ]

    def body(*refs):
        table_h = refs[:npass]
        if with_deg:
            (src_h, dst_h, z_h, zd_h, erow_h, *refs2) = refs[npass:]
            agg_o = refs2[:npass]
            (deg_o, src_v, dst_v, rows_v, acc, *rest) = refs2[npass:]
            gsem = rest[:NBUF]
            ssem = rest[NBUF:2 * NBUF]
            erow_v, dacc = rest[2 * NBUF:]
        else:
            (src_h, dst_h, z_h, *refs2) = refs[npass:]
            agg_o = refs2[:npass]
            (src_v, dst_v, rows_v, acc, *rest) = refs2[npass:]
            gsem = rest[:NBUF]
            ssem = rest[NBUF:2 * NBUF]
        c = lax.axis_index("c")
        s = lax.axis_index("s")
        w = s * NC + c
        r0 = s * ROWS_PER_TILE
        pltpu.sync_copy(src_h.at[w], src_v)
        pltpu.sync_copy(dst_h.at[w], dst_v)
        if with_deg:
            pltpu.sync_copy(zd_h.at[pl.ds(r0, ROWS_PER_TILE)],
                            dacc.at[pl.ds(r0, ROWS_PER_TILE)])
            pltpu.sync_copy(erow_h, erow_v)

        for p in range(npass):
            deg_pass = with_deg and p == 0
            pltpu.sync_copy(z_h.at[pl.ds(r0, ROWS_PER_TILE)],
                            acc.at[pl.ds(r0, ROWS_PER_TILE)])
            plsc.subcore_barrier()

            def gather(j, b, p=p):
                return pltpu.async_copy(table_h[p].at[src_v.at[j]],
                                        rows_v.at[b], gsem[b])

            for b in range(NBUF):
                gather(b, b)

            def step(t, carry, p=p, deg_pass=deg_pass, gather=gather):
                j0 = t * NBUF
                scat = []
                for b in range(NBUF):
                    j = j0 + b
                    pltpu.make_async_copy(table_h[p].at[src_v.at[j]],
                                          rows_v.at[b], gsem[b]).wait()
                    scat.append(pltpu.async_copy(
                        rows_v.at[b], acc.at[dst_v.at[j]], ssem[b], add=True))
                    if deg_pass:
                        scat.append(pltpu.async_copy(
                            erow_v, dacc.at[dst_v.at[j]], ssem[b], add=True))
                for b in range(NBUF):
                    j = j0 + NBUF + b

                    @pl.when(j < K)
                    def _():
                        nb = 2 if deg_pass else 1
                        for d in scat[b * nb:(b + 1) * nb]:
                            d.wait()
                        gather(j, b)

                return carry

            lax.fori_loop(0, K // NBUF, step, 0)
            for b in range(NBUF):
                j = K - NBUF + b
                pltpu.make_async_copy(rows_v.at[b], acc.at[dst_v.at[j]],
                                      ssem[b]).wait()
                if deg_pass:
                    pltpu.make_async_copy(erow_v, dacc.at[dst_v.at[j]],
                                          ssem[b]).wait()
            plsc.subcore_barrier()
            pltpu.sync_copy(acc.at[pl.ds(r0, ROWS_PER_TILE)],
                            agg_o[p].at[pl.ds(c * R + r0, ROWS_PER_TILE)])
        if with_deg:
            pltpu.sync_copy(dacc.at[pl.ds(r0, ROWS_PER_TILE)],
                            deg_o.at[pl.ds(c * R + r0, ROWS_PER_TILE)])

    args = list(tables) + [src3, dst3, zrows]
    if with_deg:
        args += [zdeg, erow]
    run = pl.kernel(
        body,
        out_type=out_type,
        mesh=mesh,
        scratch_types=scratch,
        compiler_params=pltpu.CompilerParams(use_tc_tiling_on_sc=False),
    )
    return run(*args)


def _tc_xr(xp, w1r, b1):
    def body(x_r, wr_r, b1_r, xr_o):
        xr_o[...] = jnp.dot(x_r[...], wr_r[...],
                            preferred_element_type=jnp.float32) + b1_r[...]

    return pl.pallas_call(
        body,
        grid=(R // BN,),
        in_specs=[
            pl.BlockSpec((BN, D_IN), lambda g: (g, 0)),
            pl.BlockSpec((D_IN, D_HID), lambda g: (0, 0)),
            pl.BlockSpec((1, D_HID), lambda g: (0, 0)),
        ],
        out_specs=pl.BlockSpec((BN, D_HID), lambda g: (g, 0)),
        out_shape=jax.ShapeDtypeStruct((R, D_HID), jnp.float32),
    )(xp, w1r, b1)


def _tc_layer1(agg0p, agg1p, degp, xr, w1l, w2l):
    HW = D_IN // 2

    def body(a0_r, a1_r, deg_r, xr_r, wl_r, w2_r, h_o, p2_o):
        deg = deg_r[0][:, 0:1] + deg_r[1][:, 0:1]
        rdeg = 1.0 / jnp.maximum(deg, 1.0)
        m0 = (a0_r[0] + a0_r[1]) * rdeg
        m1 = (a1_r[0] + a1_r[1]) * rdeg
        h = jnp.dot(m0, wl_r[0:HW], preferred_element_type=jnp.float32)
        h += jnp.dot(m1, wl_r[HW:D_IN], preferred_element_type=jnp.float32)
        h = jnp.maximum(h + xr_r[...], 0.0)
        h_o[...] = h
        p2_o[...] = jnp.dot(h, w2_r[...], preferred_element_type=jnp.float32)

    return pl.pallas_call(
        body,
        grid=(R // BN,),
        in_specs=[
            pl.BlockSpec((2, BN, HW), lambda g: (0, g, 0)),
            pl.BlockSpec((2, BN, HW), lambda g: (0, g, 0)),
            pl.BlockSpec((2, BN, 16), lambda g: (0, g, 0)),
            pl.BlockSpec((BN, D_HID), lambda g: (g, 0)),
            pl.BlockSpec((D_IN, D_HID), lambda g: (0, 0)),
            pl.BlockSpec((D_HID, D_OUT_PAD), lambda g: (0, 0)),
        ],
        out_specs=[
            pl.BlockSpec((BN, D_HID), lambda g: (g, 0)),
            pl.BlockSpec((BN, D_OUT_PAD), lambda g: (g, 0)),
        ],
        out_shape=[
            jax.ShapeDtypeStruct((R, D_HID), jnp.float32),
            jax.ShapeDtypeStruct((R, D_OUT_PAD), jnp.float32),
        ],
    )(agg0p, agg1p, degp, xr, w1l, w2l)


def _tc_hr(h, w2r, b2):
    def body(h_r, wr_r, b2_r, hr_o):
        hr_o[...] = jnp.dot(h_r[...], wr_r[...],
                            preferred_element_type=jnp.float32) + b2_r[...]

    return pl.pallas_call(
        body,
        grid=(R // BN,),
        in_specs=[
            pl.BlockSpec((BN, D_HID), lambda g: (g, 0)),
            pl.BlockSpec((D_HID, D_OUT_PAD), lambda g: (0, 0)),
            pl.BlockSpec((1, D_OUT_PAD), lambda g: (0, 0)),
        ],
        out_specs=pl.BlockSpec((BN, D_OUT_PAD), lambda g: (g, 0)),
        out_shape=jax.ShapeDtypeStruct((R, D_OUT_PAD), jnp.float32),
    )(h, w2r, b2)


def _tc_layer2(agg2p, degp, hr):
    def body(agg_r, deg_r, hr_r, out_o):
        a = agg_r[0] + agg_r[1]
        deg = deg_r[0][:, 0:1] + deg_r[1][:, 0:1]
        z = a * (1.0 / jnp.maximum(deg, 1.0)) + hr_r[...]
        col = lax.broadcasted_iota(jnp.int32, (BN, D_OUT_PAD), 1)
        z = jnp.where(col < D_OUT, z, -1e30)
        m = jnp.max(z, axis=-1, keepdims=True)
        e = jnp.exp(z - m)
        lse = jnp.log(jnp.sum(e, axis=-1, keepdims=True))
        out_o[...] = z - m - lse

    return pl.pallas_call(
        body,
        grid=(R // BN,),
        in_specs=[
            pl.BlockSpec((2, BN, D_OUT_PAD), lambda g: (0, g, 0)),
            pl.BlockSpec((2, BN, 16), lambda g: (0, g, 0)),
            pl.BlockSpec((BN, D_OUT_PAD), lambda g: (g, 0)),
        ],
        out_specs=pl.BlockSpec((BN, D_OUT_PAD), lambda g: (g, 0)),
        out_shape=jax.ShapeDtypeStruct((R, D_OUT_PAD), jnp.float32),
    )(agg2p, degp, hr)


def kernel(x, edge_index, W1_l, W1_r, b1, W2_l, W2_r, b2):
    src = edge_index[0].astype(jnp.int32)
    dst = edge_index[1].astype(jnp.int32)
    npad = EP - N_EDGES
    pad_i = jnp.arange(npad, dtype=jnp.int32)
    src3 = jnp.concatenate([src, pad_i % N_NODES]).reshape(NW, K, EB)
    dst3 = jnp.concatenate([dst, N_NODES + pad_i % (R - N_NODES)]).reshape(NW, K, EB)

    xp = jnp.pad(x, ((0, R - N_NODES), (0, 0)))
    w2l = jnp.pad(W2_l, ((0, 0), (0, D_OUT_PAD - D_OUT)))
    w2r = jnp.pad(W2_r, ((0, 0), (0, D_OUT_PAD - D_OUT)))
    b1r = b1.reshape(1, D_HID)
    b2r = jnp.pad(b2, (0, D_OUT_PAD - D_OUT)).reshape(1, D_OUT_PAD)

    HW = D_IN // 2
    zx = jnp.zeros((R, HW), jnp.float32)
    zd = jnp.zeros((R, 16), jnp.float32)
    z2 = jnp.zeros((R, D_OUT_PAD), jnp.float32)
    erow = jnp.zeros((EB, 16), jnp.float32).at[:, 0].set(1.0)

    xh0 = xp[:, :HW]
    xh1 = xp[:, HW:]
    agg0p, agg1p, degp = _sc_aggregate(
        [xh0, xh1], src3, dst3, zx, HW, True, 5, zdeg=zd, erow=erow)
    xr = _tc_xr(xp, W1_r, b1r)
    agg0p = agg0p.reshape(NC, R, HW)
    agg1p = agg1p.reshape(NC, R, HW)
    degp = degp.reshape(NC, R, 16)

    h, p2 = _tc_layer1(agg0p, agg1p, degp, xr, W1_l, w2l)

    (agg2p,) = _sc_aggregate([p2], src3, dst3, z2, D_OUT_PAD, False, 8)
    hr = _tc_hr(h, w2r, b2r)
    agg2p = agg2p.reshape(NC, R, D_OUT_PAD)

    out = _tc_layer2(agg2p, degp, hr)
    return out[:N_NODES, :D_OUT]


# speedup vs baseline: 16.6585x; 1.1054x over previous
import functools

import jax
import jax.numpy as jnp
from jax import lax
from jax.experimental import pallas as pl
from jax.experimental.pallas import tpu as pltpu
from jax.experimental.pallas import tpu_sc as plsc

N_NODES = 10000
N_EDGES = 320000
D_IN = 128
D_HID = 256
D_OUT = 40
D_OUT_PAD = 48

NC = 2
NS = 16
NW = NC * NS

R = 10240
ROWS_PER_TILE = R // NS
EB = 128
EP = 327680
K = EP // (NW * EB)

BN = 1024


def _sc_aggregate(tables, src3, dst3, zrows, width, with_deg, nbuf, zdeg=None, erow=None):
    npass = len(tables)
    NBUF = nbuf
    mesh = plsc.VectorSubcoreMesh(
        core_axis_name="c", subcore_axis_name="s", num_cores=NC, num_subcores=NS
    )
    out_type = [jax.ShapeDtypeStruct((NC * R, npass * width), jnp.float32)]
    scratch = [
        pltpu.VMEM((K, EB), jnp.int32),
        pltpu.VMEM((K, EB), jnp.int32),
        pltpu.VMEM((NBUF, EB, width), jnp.float32),
        pltpu.VMEM_SHARED((R, width), jnp.float32),
    ]
    scratch += [pltpu.SemaphoreType.DMA] * (2 * NBUF)
    if with_deg:
        out_type.append(jax.ShapeDtypeStruct((NC * R, 16), jnp.float32))
        scratch += [
            pltpu.VMEM((EB, 16), jnp.float32),
            pltpu.VMEM_SHARED((R, 16), jnp.float32),
        ]

    def body(*refs):
        table_h = refs[:npass]
        if with_deg:
            (src_h, dst_h, z_h, zd_h, erow_h, agg_o, deg_o,
             src_v, dst_v, rows_v, acc, *rest) = refs[npass:]
            gsem = rest[:NBUF]
            ssem = rest[NBUF:2 * NBUF]
            erow_v, dacc = rest[2 * NBUF:]
        else:
            (src_h, dst_h, z_h, agg_o,
             src_v, dst_v, rows_v, acc, *rest) = refs[npass:]
            gsem = rest[:NBUF]
            ssem = rest[NBUF:2 * NBUF]
        c = lax.axis_index("c")
        s = lax.axis_index("s")
        w = s * NC + c
        r0 = s * ROWS_PER_TILE
        pltpu.sync_copy(src_h.at[w], src_v)
        pltpu.sync_copy(dst_h.at[w], dst_v)
        if with_deg:
            pltpu.sync_copy(zd_h.at[pl.ds(r0, ROWS_PER_TILE)],
                            dacc.at[pl.ds(r0, ROWS_PER_TILE)])
            pltpu.sync_copy(erow_h, erow_v)

        for p in range(npass):
            deg_pass = with_deg and p == 0
            pltpu.sync_copy(z_h.at[pl.ds(r0, ROWS_PER_TILE)],
                            acc.at[pl.ds(r0, ROWS_PER_TILE)])
            plsc.subcore_barrier()

            def gather(j, b, p=p):
                return pltpu.async_copy(table_h[p].at[src_v.at[j]],
                                        rows_v.at[b], gsem[b])

            for b in range(NBUF):
                gather(b, b)

            def step(t, carry, p=p, deg_pass=deg_pass, gather=gather):
                j0 = t * NBUF
                scat = []
                for b in range(NBUF):
                    j = j0 + b
                    pltpu.make_async_copy(table_h[p].at[src_v.at[j]],
                                          rows_v.at[b], gsem[b]).wait()
                    scat.append(pltpu.async_copy(
                        rows_v.at[b], acc.at[dst_v.at[j]], ssem[b], add=True))
                    if deg_pass:
                        scat.append(pltpu.async_copy(
                            erow_v, dacc.at[dst_v.at[j]], ssem[b], add=True))
                for b in range(NBUF):
                    j = j0 + NBUF + b

                    @pl.when(j < K)
                    def _():
                        nb = 2 if deg_pass else 1
                        for d in scat[b * nb:(b + 1) * nb]:
                            d.wait()
                        gather(j, b)

                return carry

            lax.fori_loop(0, K // NBUF, step, 0)
            for b in range(NBUF):
                j = K - NBUF + b
                pltpu.make_async_copy(rows_v.at[b], acc.at[dst_v.at[j]],
                                      ssem[b]).wait()
                if deg_pass:
                    pltpu.make_async_copy(erow_v, dacc.at[dst_v.at[j]],
                                          ssem[b]).wait()
            plsc.subcore_barrier()
            if npass == 1:
                pltpu.sync_copy(acc.at[pl.ds(r0, ROWS_PER_TILE)],
                                agg_o.at[pl.ds(c * R + r0, ROWS_PER_TILE)])
            else:
                pltpu.sync_copy(
                    acc.at[pl.ds(r0, ROWS_PER_TILE)],
                    agg_o.at[pl.ds(c * R + r0, ROWS_PER_TILE),
                             pl.ds(p * width, width)])
        if with_deg:
            pltpu.sync_copy(dacc.at[pl.ds(r0, ROWS_PER_TILE)],
                            deg_o.at[pl.ds(c * R + r0, ROWS_PER_TILE)])

    args = list(tables) + [src3, dst3, zrows]
    if with_deg:
        args += [zdeg, erow]
    run = pl.kernel(
        body,
        out_type=out_type,
        mesh=mesh,
        scratch_types=scratch,
        compiler_params=pltpu.CompilerParams(use_tc_tiling_on_sc=False),
    )
    return run(*args)


def _tc_edge_prep(edge_index):
    ROWS = EP // EB
    EROWS = N_EDGES // EB

    def body(e_r, s_o, d_o):
        e = e_r[...].reshape(2, EROWS, EB)
        r = (lax.broadcasted_iota(jnp.int32, (ROWS - EROWS, EB), 0) * EB
             + lax.broadcasted_iota(jnp.int32, (ROWS - EROWS, EB), 1))
        s_o[...] = jnp.concatenate([e[0], r % N_NODES], axis=0)
        d_o[...] = jnp.concatenate([e[1], N_NODES + r % (R - N_NODES)], axis=0)

    return pl.pallas_call(
        body,
        grid=(1,),
        in_specs=[pl.BlockSpec((2, N_EDGES), lambda g: (0, 0))],
        out_specs=[
            pl.BlockSpec((ROWS, EB), lambda g: (0, 0)),
            pl.BlockSpec((ROWS, EB), lambda g: (0, 0)),
        ],
        out_shape=[
            jax.ShapeDtypeStruct((ROWS, EB), jnp.int32),
            jax.ShapeDtypeStruct((ROWS, EB), jnp.int32),
        ],
    )(edge_index)


def _tc_xr(xp, w1r, b1):
    def body(x_r, wr_r, b1_r, xr_o):
        xr_o[...] = jnp.dot(x_r[...], wr_r[...],
                            preferred_element_type=jnp.float32) + b1_r[...]

    return pl.pallas_call(
        body,
        grid=(R // BN,),
        in_specs=[
            pl.BlockSpec((BN, D_IN), lambda g: (g, 0)),
            pl.BlockSpec((D_IN, D_HID), lambda g: (0, 0)),
            pl.BlockSpec((1, D_HID), lambda g: (0, 0)),
        ],
        out_specs=pl.BlockSpec((BN, D_HID), lambda g: (g, 0)),
        out_shape=jax.ShapeDtypeStruct((R, D_HID), jnp.float32),
    )(xp, w1r, b1)


def _tc_layer1(aggp, degp, xr, w1l, w2l):
    def body(a_r, deg_r, xr_r, wl_r, w2_r, h_o, p2_o):
        deg = deg_r[0][:, 0:1] + deg_r[1][:, 0:1]
        rdeg = 1.0 / jnp.maximum(deg, 1.0)
        m = (a_r[0] + a_r[1]) * rdeg
        h = jnp.dot(m, wl_r[...], preferred_element_type=jnp.float32)
        h = jnp.maximum(h + xr_r[...], 0.0)
        h_o[...] = h
        p2_o[...] = jnp.dot(h, w2_r[...], preferred_element_type=jnp.float32)

    return pl.pallas_call(
        body,
        grid=(R // BN,),
        in_specs=[
            pl.BlockSpec((2, BN, D_IN), lambda g: (0, g, 0)),
            pl.BlockSpec((2, BN, 16), lambda g: (0, g, 0)),
            pl.BlockSpec((BN, D_HID), lambda g: (g, 0)),
            pl.BlockSpec((D_IN, D_HID), lambda g: (0, 0)),
            pl.BlockSpec((D_HID, D_OUT_PAD), lambda g: (0, 0)),
        ],
        out_specs=[
            pl.BlockSpec((BN, D_HID), lambda g: (g, 0)),
            pl.BlockSpec((BN, D_OUT_PAD), lambda g: (g, 0)),
        ],
        out_shape=[
            jax.ShapeDtypeStruct((R, D_HID), jnp.float32),
            jax.ShapeDtypeStruct((R, D_OUT_PAD), jnp.float32),
        ],
    )(aggp, degp, xr, w1l, w2l)


def _tc_hr(h, w2r, b2):
    def body(h_r, wr_r, b2_r, hr_o):
        hr_o[...] = jnp.dot(h_r[...], wr_r[...],
                            preferred_element_type=jnp.float32) + b2_r[...]

    return pl.pallas_call(
        body,
        grid=(R // BN,),
        in_specs=[
            pl.BlockSpec((BN, D_HID), lambda g: (g, 0)),
            pl.BlockSpec((D_HID, D_OUT_PAD), lambda g: (0, 0)),
            pl.BlockSpec((1, D_OUT_PAD), lambda g: (0, 0)),
        ],
        out_specs=pl.BlockSpec((BN, D_OUT_PAD), lambda g: (g, 0)),
        out_shape=jax.ShapeDtypeStruct((R, D_OUT_PAD), jnp.float32),
    )(h, w2r, b2)


def _tc_layer2(agg2p, degp, hr):
    def body(agg_r, deg_r, hr_r, out_o):
        a = agg_r[0] + agg_r[1]
        deg = deg_r[0][:, 0:1] + deg_r[1][:, 0:1]
        z = a * (1.0 / jnp.maximum(deg, 1.0)) + hr_r[...]
        col = lax.broadcasted_iota(jnp.int32, (BN, D_OUT_PAD), 1)
        z = jnp.where(col < D_OUT, z, -1e30)
        m = jnp.max(z, axis=-1, keepdims=True)
        e = jnp.exp(z - m)
        lse = jnp.log(jnp.sum(e, axis=-1, keepdims=True))
        out_o[...] = z - m - lse

    return pl.pallas_call(
        body,
        grid=(R // BN,),
        in_specs=[
            pl.BlockSpec((2, BN, D_OUT_PAD), lambda g: (0, g, 0)),
            pl.BlockSpec((2, BN, 16), lambda g: (0, g, 0)),
            pl.BlockSpec((BN, D_OUT_PAD), lambda g: (g, 0)),
        ],
        out_specs=pl.BlockSpec((BN, D_OUT_PAD), lambda g: (g, 0)),
        out_shape=jax.ShapeDtypeStruct((R, D_OUT_PAD), jnp.float32),
    )(agg2p, degp, hr)


def kernel(x, edge_index, W1_l, W1_r, b1, W2_l, W2_r, b2):
    s_flat, d_flat = _tc_edge_prep(edge_index.astype(jnp.int32))
    src3 = s_flat.reshape(NW, K, EB)
    dst3 = d_flat.reshape(NW, K, EB)

    xp = jnp.pad(x, ((0, R - N_NODES), (0, 0)))
    w2l = jnp.pad(W2_l, ((0, 0), (0, D_OUT_PAD - D_OUT)))
    w2r = jnp.pad(W2_r, ((0, 0), (0, D_OUT_PAD - D_OUT)))
    b1r = b1.reshape(1, D_HID)
    b2r = jnp.pad(b2, (0, D_OUT_PAD - D_OUT)).reshape(1, D_OUT_PAD)

    HW = D_IN // 2
    zx = jnp.zeros((R, HW), jnp.float32)
    zd = jnp.zeros((R, 16), jnp.float32)
    z2 = jnp.zeros((R, D_OUT_PAD), jnp.float32)
    erow = jnp.zeros((EB, 16), jnp.float32).at[:, 0].set(1.0)

    xh0 = xp[:, :HW]
    xh1 = xp[:, HW:]
    aggp, degp = _sc_aggregate(
        [xh0, xh1], src3, dst3, zx, HW, True, 5, zdeg=zd, erow=erow)
    xr = _tc_xr(xp, W1_r, b1r)
    aggp = aggp.reshape(NC, R, D_IN)
    degp = degp.reshape(NC, R, 16)

    h, p2 = _tc_layer1(aggp, degp, xr, W1_l, w2l)

    (agg2p,) = _sc_aggregate([p2], src3, dst3, z2, D_OUT_PAD, False, 8)
    hr = _tc_hr(h, w2r, b2r)
    agg2p = agg2p.reshape(NC, R, D_OUT_PAD)

    out = _tc_layer2(agg2p, degp, hr)
    return out[:N_NODES, :D_OUT]


# speedup vs baseline: 18.1915x; 1.0920x over previous
import jax
import jax.numpy as jnp
from jax import lax
from jax.experimental import pallas as pl
from jax.experimental.pallas import tpu as pltpu
from jax.experimental.pallas import tpu_sc as plsc

N_NODES = 10000
N_EDGES = 320000
D_IN = 128
D_HID = 256
D_OUT = 40
D_OUT_PAD = 48

NC = 2
NS = 16
NW = NC * NS

R = 10240
RPT = R // NS
EB = 128
EP = 327680
K = EP // (NW * EB)
EROWS = EP // EB

BN = 1024
NG = (N_NODES + BN - 1) // BN


def _sc_aggregate(tables, srci, dsti, zrows, width, with_deg, nbuf,
                  zdeg=None, erow=None):
    npass = len(tables)
    mesh = plsc.VectorSubcoreMesh(
        core_axis_name="c", subcore_axis_name="s", num_cores=NC, num_subcores=NS
    )
    out_type = [jax.ShapeDtypeStruct((NC * R, 128), jnp.float32)]
    scratch = [
        pltpu.VMEM((K, EB), jnp.int32),
        pltpu.VMEM((K, EB), jnp.int32),
        pltpu.VMEM((nbuf, EB, width), jnp.float32),
        pltpu.VMEM_SHARED((R, width), jnp.float32),
    ]
    scratch += [pltpu.SemaphoreType.DMA] * (2 * nbuf)
    if with_deg:
        out_type.append(jax.ShapeDtypeStruct((NC * R, 16), jnp.float32))
        scratch += [
            pltpu.VMEM((EB, 16), jnp.float32),
            pltpu.VMEM_SHARED((R, 16), jnp.float32),
        ]

    def body(*refs):
        table_h = refs[:npass]
        if with_deg:
            (src_h, dst_h, z_h, zd_h, erow_h, agg_o, deg_o,
             src_v, dst_v, rows_v, acc, *rest) = refs[npass:]
            gsem = rest[:nbuf]
            ssem = rest[nbuf:2 * nbuf]
            erow_v, dacc = rest[2 * nbuf:]
        else:
            (src_h, dst_h, z_h, agg_o,
             src_v, dst_v, rows_v, acc, *rest) = refs[npass:]
            gsem = rest[:nbuf]
            ssem = rest[nbuf:2 * nbuf]
        c = lax.axis_index("c")
        s = lax.axis_index("s")
        w = s * NC + c
        r0 = s * RPT
        pltpu.sync_copy(src_h.at[pl.ds(w * K, K)], src_v)
        pltpu.sync_copy(dst_h.at[pl.ds(w * K, K)], dst_v)
        if with_deg:
            pltpu.sync_copy(zd_h, dacc.at[pl.ds(r0, RPT)])
            pltpu.sync_copy(erow_h, erow_v)

        for p in range(npass):
            deg_pass = with_deg and p == 0
            pltpu.sync_copy(z_h, acc.at[pl.ds(r0, RPT)])
            plsc.subcore_barrier()

            def gather(j, b, p=p):
                return pltpu.async_copy(table_h[p].at[src_v.at[j]],
                                        rows_v.at[b], gsem[b])

            for b in range(nbuf):
                gather(b, b)

            def step(t, carry, p=p, deg_pass=deg_pass, gather=gather):
                j0 = t * nbuf
                scat = []
                for b in range(nbuf):
                    j = j0 + b
                    pltpu.make_async_copy(table_h[p].at[src_v.at[j]],
                                          rows_v.at[b], gsem[b]).wait()
                    scat.append(pltpu.async_copy(
                        rows_v.at[b], acc.at[dst_v.at[j]], ssem[b], add=True))
                    if deg_pass:
                        scat.append(pltpu.async_copy(
                            erow_v, dacc.at[dst_v.at[j]], ssem[b], add=True))
                for b in range(nbuf):
                    j = j0 + nbuf + b

                    @pl.when(j < K)
                    def _():
                        nb = 2 if deg_pass else 1
                        for d in scat[b * nb:(b + 1) * nb]:
                            d.wait()
                        gather(j, b)

                return carry

            lax.fori_loop(0, K // nbuf, step, 0)
            for b in range(nbuf):
                j = K - nbuf + b
                pltpu.make_async_copy(rows_v.at[b], acc.at[dst_v.at[j]],
                                      ssem[b]).wait()
                if deg_pass:
                    pltpu.make_async_copy(erow_v, dacc.at[dst_v.at[j]],
                                          ssem[b]).wait()
            plsc.subcore_barrier()
            pltpu.sync_copy(
                acc.at[pl.ds(r0, RPT)],
                agg_o.at[pl.ds(c * R + r0, RPT), pl.ds(p * width, width)])
        if with_deg:
            pltpu.sync_copy(dacc.at[pl.ds(r0, RPT)],
                            deg_o.at[pl.ds(c * R + r0, RPT)])

    args = list(tables) + [srci, dsti, zrows]
    if with_deg:
        args += [zdeg, erow]
    run = pl.kernel(
        body,
        out_type=out_type,
        mesh=mesh,
        scratch_types=scratch,
        compiler_params=pltpu.CompilerParams(use_tc_tiling_on_sc=False),
    )
    return run(*args)


def _tc_edge_prep(edge_index):
    ER = N_EDGES // EB

    def body(e_r, s_o, d_o):
        e = e_r[...]
        e0 = e[0].reshape(ER, EB)
        e1 = e[1].reshape(ER, EB)
        r = (lax.broadcasted_iota(jnp.int32, (EROWS - ER, EB), 0) * EB
             + lax.broadcasted_iota(jnp.int32, (EROWS - ER, EB), 1))
        s_o[...] = jnp.concatenate([e0, r % N_NODES], axis=0)
        d_o[...] = jnp.concatenate([e1, N_NODES + r % (R - N_NODES)], axis=0)

    return pl.pallas_call(
        body,
        grid=(1,),
        in_specs=[pl.BlockSpec((2, N_EDGES), lambda g: (0, 0))],
        out_specs=[
            pl.BlockSpec((EROWS, EB), lambda g: (0, 0)),
            pl.BlockSpec((EROWS, EB), lambda g: (0, 0)),
        ],
        out_shape=[
            jax.ShapeDtypeStruct((EROWS, EB), jnp.int32),
            jax.ShapeDtypeStruct((EROWS, EB), jnp.int32),
        ],
    )(edge_index)


def _apply_rdeg(a, d0, d1):
    d = (d0 + d1).reshape(BN // 8, 8, 16)
    rdeg = 1.0 / jnp.maximum(d[:, :, 0:1], 1.0)
    w = a.shape[-1]
    return (a.reshape(BN // 8, 8, w) * rdeg).reshape(BN, w)


def _tc_layer1(aggp, degp, x, w1l, w1r, b1, w2l, w2r, b2):
    def body(a0_r, a1_r, d0_r, d1_r, x_r, wl_r, wr_r, b1_r, w2l_r, w2r_r,
             b2_r, p2_o, hr_o):
        m = _apply_rdeg(a0_r[...] + a1_r[...], d0_r[...], d1_r[...])
        h = jnp.dot(m, wl_r[...], preferred_element_type=jnp.float32)
        h += jnp.dot(x_r[...], wr_r[...], preferred_element_type=jnp.float32)
        h = jnp.maximum(h + b1_r[...], 0.0)
        p2_o[...] = jnp.dot(h, w2l_r[...], preferred_element_type=jnp.float32)
        hr_o[...] = jnp.dot(h, w2r_r[...],
                            preferred_element_type=jnp.float32) + b2_r[...]

    GB = R // BN
    DB = BN // 8
    return pl.pallas_call(
        body,
        grid=(NG,),
        in_specs=[
            pl.BlockSpec((BN, D_IN), lambda g: (g, 0)),
            pl.BlockSpec((BN, D_IN), lambda g: (GB + g, 0)),
            pl.BlockSpec((DB, 128), lambda g: (g, 0)),
            pl.BlockSpec((DB, 128), lambda g: (GB + g, 0)),
            pl.BlockSpec((BN, D_IN), lambda g: (g, 0)),
            pl.BlockSpec((D_IN, D_HID), lambda g: (0, 0)),
            pl.BlockSpec((D_IN, D_HID), lambda g: (0, 0)),
            pl.BlockSpec((1, D_HID), lambda g: (0, 0)),
            pl.BlockSpec((D_HID, D_OUT_PAD), lambda g: (0, 0)),
            pl.BlockSpec((D_HID, D_OUT_PAD), lambda g: (0, 0)),
            pl.BlockSpec((1, D_OUT_PAD), lambda g: (0, 0)),
        ],
        out_specs=[
            pl.BlockSpec((BN, D_OUT_PAD), lambda g: (g, 0)),
            pl.BlockSpec((BN, D_OUT_PAD), lambda g: (g, 0)),
        ],
        out_shape=[
            jax.ShapeDtypeStruct((N_NODES, D_OUT_PAD), jnp.float32),
            jax.ShapeDtypeStruct((N_NODES, D_OUT_PAD), jnp.float32),
        ],
    )(aggp, aggp, degp, degp, x, w1l, w1r, b1, w2l, w2r, b2)


def _tc_layer2(agg2p, degp, hr):
    def body(a0_r, a1_r, d0_r, d1_r, hr_r, out_o):
        a = a0_r[...][:, 0:D_OUT_PAD] + a1_r[...][:, 0:D_OUT_PAD]
        z = _apply_rdeg(a, d0_r[...], d1_r[...]) + hr_r[...]
        col = lax.broadcasted_iota(jnp.int32, (BN, D_OUT_PAD), 1)
        z = jnp.where(col < D_OUT, z, -1e30)
        m = jnp.max(z, axis=-1, keepdims=True)
        e = jnp.exp(z - m)
        lse = jnp.log(jnp.sum(e, axis=-1, keepdims=True))
        out_o[...] = (z - m - lse)[:, 0:D_OUT]

    GB = R // BN
    DB = BN // 8
    return pl.pallas_call(
        body,
        grid=(NG,),
        in_specs=[
            pl.BlockSpec((BN, 128), lambda g: (g, 0)),
            pl.BlockSpec((BN, 128), lambda g: (GB + g, 0)),
            pl.BlockSpec((DB, 128), lambda g: (g, 0)),
            pl.BlockSpec((DB, 128), lambda g: (GB + g, 0)),
            pl.BlockSpec((BN, D_OUT_PAD), lambda g: (g, 0)),
        ],
        out_specs=pl.BlockSpec((BN, D_OUT), lambda g: (g, 0)),
        out_shape=jax.ShapeDtypeStruct((N_NODES, D_OUT), jnp.float32),
    )(agg2p, agg2p, degp, degp, hr)


def kernel(x, edge_index, W1_l, W1_r, b1, W2_l, W2_r, b2):
    s2, d2 = _tc_edge_prep(edge_index.astype(jnp.int32))

    w2l = jnp.pad(W2_l, ((0, 0), (0, D_OUT_PAD - D_OUT)))
    w2r = jnp.pad(W2_r, ((0, 0), (0, D_OUT_PAD - D_OUT)))
    b1r = b1.reshape(1, D_HID)
    b2r = jnp.pad(b2, (0, D_OUT_PAD - D_OUT)).reshape(1, D_OUT_PAD)

    HW = D_IN // 2
    zx = jnp.zeros((RPT, HW), jnp.float32)
    zd = jnp.zeros((RPT, 16), jnp.float32)
    z2 = jnp.zeros((RPT, D_OUT_PAD), jnp.float32)
    erow = jnp.zeros((EB, 16), jnp.float32).at[:, 0].set(1.0)

    xh0 = x[:, :HW]
    xh1 = x[:, HW:]
    aggp, degflat = _sc_aggregate(
        [xh0, xh1], s2, d2, zx, HW, True, 5, zdeg=zd, erow=erow)
    degp = degflat.reshape(NC * R // 8, 128)

    p2, hr = _tc_layer1(aggp, degp, x, W1_l, W1_r, b1r, w2l, w2r, b2r)

    (agg2p,) = _sc_aggregate([p2], s2, d2, z2, D_OUT_PAD, False, 8)

    return _tc_layer2(agg2p, degp, hr)


# speedup vs baseline: 19.0927x; 1.0495x over previous
import jax
import jax.numpy as jnp
from jax import lax
from jax.experimental import pallas as pl
from jax.experimental.pallas import tpu as pltpu
from jax.experimental.pallas import tpu_sc as plsc

N_NODES = 10000
N_EDGES = 320000
D_IN = 128
D_HID = 256
D_OUT = 40
D_OUT_PAD = 48

NC = 2
NS = 16
NW = NC * NS

R = 10240
RPT = R // NS
EB = 128
EP = 327680
K = EP // (NW * EB)
EROWS = EP // EB

BN = 1024
NG = (N_NODES + BN - 1) // BN


def _sc_aggregate(tables, srcs, dsti, zrows, width, with_deg, nbuf,
                  zdeg=None, erow=None):
    npass = len(tables)
    assert len(srcs) == npass
    mesh = plsc.VectorSubcoreMesh(
        core_axis_name="c", subcore_axis_name="s", num_cores=NC, num_subcores=NS
    )
    out_type = [jax.ShapeDtypeStruct((NC * R, 128), jnp.float32)]
    scratch = [pltpu.VMEM((K, EB), jnp.int32) for _ in range(npass)]
    scratch += [
        pltpu.VMEM((K, EB), jnp.int32),
        pltpu.VMEM((nbuf, EB, width), jnp.float32),
        pltpu.VMEM_SHARED((R, width), jnp.float32),
    ]
    scratch += [pltpu.SemaphoreType.DMA] * (2 * nbuf)
    if with_deg:
        out_type.append(jax.ShapeDtypeStruct((NC * R, 16), jnp.float32))
        scratch += [
            pltpu.VMEM((EB, 16), jnp.float32),
            pltpu.VMEM_SHARED((R, 16), jnp.float32),
        ]

    def body(*refs):
        table_h = refs[:npass]
        src_h = refs[npass:2 * npass]
        if with_deg:
            (dst_h, z_h, zd_h, erow_h, agg_o, deg_o, *refs2) = refs[2 * npass:]
            src_v = refs2[:npass]
            (dst_v, rows_v, acc, *rest) = refs2[npass:]
            gsem = rest[:nbuf]
            ssem = rest[nbuf:2 * nbuf]
            erow_v, dacc = rest[2 * nbuf:]
        else:
            (dst_h, z_h, agg_o, *refs2) = refs[2 * npass:]
            src_v = refs2[:npass]
            (dst_v, rows_v, acc, *rest) = refs2[npass:]
            gsem = rest[:nbuf]
            ssem = rest[nbuf:2 * nbuf]
        c = lax.axis_index("c")
        s = lax.axis_index("s")
        w = s * NC + c
        r0 = s * RPT
        for p in range(npass):
            pltpu.sync_copy(src_h[p].at[pl.ds(w * K, K)], src_v[p])
        pltpu.sync_copy(dst_h.at[pl.ds(w * K, K)], dst_v)
        if with_deg:
            pltpu.sync_copy(zd_h, dacc.at[pl.ds(r0, RPT)])
            pltpu.sync_copy(erow_h, erow_v)

        for p in range(npass):
            deg_pass = with_deg and p == 0
            pltpu.sync_copy(z_h, acc.at[pl.ds(r0, RPT)])
            plsc.subcore_barrier()

            def gather(j, b, p=p):
                return pltpu.async_copy(table_h[p].at[src_v[p].at[j]],
                                        rows_v.at[b], gsem[b])

            for b in range(nbuf):
                gather(b, b)

            def step(t, carry, p=p, deg_pass=deg_pass, gather=gather):
                j0 = t * nbuf
                scat = []
                for b in range(nbuf):
                    j = j0 + b
                    pltpu.make_async_copy(table_h[p].at[src_v[p].at[j]],
                                          rows_v.at[b], gsem[b]).wait()
                    scat.append(pltpu.async_copy(
                        rows_v.at[b], acc.at[dst_v.at[j]], ssem[b], add=True))
                    if deg_pass:
                        scat.append(pltpu.async_copy(
                            erow_v, dacc.at[dst_v.at[j]], ssem[b], add=True))
                for b in range(nbuf):
                    j = j0 + nbuf + b

                    @pl.when(j < K)
                    def _():
                        nb = 2 if deg_pass else 1
                        for d in scat[b * nb:(b + 1) * nb]:
                            d.wait()
                        gather(j, b)

                return carry

            lax.fori_loop(0, K // nbuf, step, 0)
            for b in range(nbuf):
                j = K - nbuf + b
                pltpu.make_async_copy(rows_v.at[b], acc.at[dst_v.at[j]],
                                      ssem[b]).wait()
                if deg_pass:
                    pltpu.make_async_copy(erow_v, dacc.at[dst_v.at[j]],
                                          ssem[b]).wait()
            plsc.subcore_barrier()
            pltpu.sync_copy(
                acc.at[pl.ds(r0, RPT)],
                agg_o.at[pl.ds(c * R + r0, RPT), pl.ds(p * width, width)])
        if with_deg:
            pltpu.sync_copy(dacc.at[pl.ds(r0, RPT)],
                            deg_o.at[pl.ds(c * R + r0, RPT)])

    args = list(tables) + list(srcs) + [dsti, zrows]
    if with_deg:
        args += [zdeg, erow]
    run = pl.kernel(
        body,
        out_type=out_type,
        mesh=mesh,
        scratch_types=scratch,
        compiler_params=pltpu.CompilerParams(use_tc_tiling_on_sc=False),
    )
    return run(*args)


def _tc_edge_prep(edge_index):
    ER = N_EDGES // EB

    def body(e_r, s_o, d_o, sa_o, sb_o):
        e = e_r[...]
        e0 = e[0].reshape(ER, EB)
        e1 = e[1].reshape(ER, EB)
        r = (lax.broadcasted_iota(jnp.int32, (EROWS - ER, EB), 0) * EB
             + lax.broadcasted_iota(jnp.int32, (EROWS - ER, EB), 1))
        s = jnp.concatenate([e0, r % N_NODES], axis=0)
        s_o[...] = s
        d_o[...] = jnp.concatenate([e1, N_NODES + r % (R - N_NODES)], axis=0)
        sa_o[...] = 2 * s
        sb_o[...] = 2 * s + 1

    return pl.pallas_call(
        body,
        grid=(1,),
        in_specs=[pl.BlockSpec((2, N_EDGES), lambda g: (0, 0))],
        out_specs=[pl.BlockSpec((EROWS, EB), lambda g: (0, 0))] * 4,
        out_shape=[jax.ShapeDtypeStruct((EROWS, EB), jnp.int32)] * 4,
    )(edge_index)


def _apply_rdeg(a, d0, d1):
    d = (d0 + d1).reshape(BN // 8, 8, 16)
    rdeg = 1.0 / jnp.maximum(d[:, :, 0:1], 1.0)
    w = a.shape[-1]
    return (a.reshape(BN // 8, 8, w) * rdeg).reshape(BN, w)


def _tc_layer1(aggp, degp, x, w1l, w1r, b1, w2l, w2r, b2):
    def body(a0_r, a1_r, d0_r, d1_r, x_r, wl_r, wr_r, b1_r, w2l_r, w2r_r,
             b2_r, p2_o, hr_o):
        m = _apply_rdeg(a0_r[...] + a1_r[...], d0_r[...], d1_r[...])
        h = jnp.dot(m, wl_r[...], preferred_element_type=jnp.float32)
        h += jnp.dot(x_r[...], wr_r[...], preferred_element_type=jnp.float32)
        h = jnp.maximum(h + b1_r[...], 0.0)
        p2_o[...] = jnp.dot(h, w2l_r[...], preferred_element_type=jnp.float32)
        hr_o[...] = jnp.dot(h, w2r_r[...],
                            preferred_element_type=jnp.float32) + b2_r[...]

    GB = R // BN
    DB = BN // 8
    return pl.pallas_call(
        body,
        grid=(NG,),
        in_specs=[
            pl.BlockSpec((BN, D_IN), lambda g: (g, 0)),
            pl.BlockSpec((BN, D_IN), lambda g: (GB + g, 0)),
            pl.BlockSpec((DB, 128), lambda g: (g, 0)),
            pl.BlockSpec((DB, 128), lambda g: (GB + g, 0)),
            pl.BlockSpec((BN, D_IN), lambda g: (g, 0)),
            pl.BlockSpec((D_IN, D_HID), lambda g: (0, 0)),
            pl.BlockSpec((D_IN, D_HID), lambda g: (0, 0)),
            pl.BlockSpec((1, D_HID), lambda g: (0, 0)),
            pl.BlockSpec((D_HID, D_OUT_PAD), lambda g: (0, 0)),
            pl.BlockSpec((D_HID, D_OUT_PAD), lambda g: (0, 0)),
            pl.BlockSpec((1, D_OUT_PAD), lambda g: (0, 0)),
        ],
        out_specs=[
            pl.BlockSpec((BN, D_OUT_PAD), lambda g: (g, 0)),
            pl.BlockSpec((BN, D_OUT_PAD), lambda g: (g, 0)),
        ],
        out_shape=[
            jax.ShapeDtypeStruct((N_NODES, D_OUT_PAD), jnp.float32),
            jax.ShapeDtypeStruct((N_NODES, D_OUT_PAD), jnp.float32),
        ],
    )(aggp, aggp, degp, degp, x, w1l, w1r, b1, w2l, w2r, b2)


def _tc_layer2(agg2p, degp, hr):
    def body(a0_r, a1_r, d0_r, d1_r, hr_r, out_o):
        a = a0_r[...][:, 0:D_OUT_PAD] + a1_r[...][:, 0:D_OUT_PAD]
        z = _apply_rdeg(a, d0_r[...], d1_r[...]) + hr_r[...]
        col = lax.broadcasted_iota(jnp.int32, (BN, D_OUT_PAD), 1)
        z = jnp.where(col < D_OUT, z, -1e30)
        m = jnp.max(z, axis=-1, keepdims=True)
        e = jnp.exp(z - m)
        lse = jnp.log(jnp.sum(e, axis=-1, keepdims=True))
        out_o[...] = (z - m - lse)[:, 0:D_OUT]

    GB = R // BN
    DB = BN // 8
    return pl.pallas_call(
        body,
        grid=(NG,),
        in_specs=[
            pl.BlockSpec((BN, 128), lambda g: (g, 0)),
            pl.BlockSpec((BN, 128), lambda g: (GB + g, 0)),
            pl.BlockSpec((DB, 128), lambda g: (g, 0)),
            pl.BlockSpec((DB, 128), lambda g: (GB + g, 0)),
            pl.BlockSpec((BN, D_OUT_PAD), lambda g: (g, 0)),
        ],
        out_specs=pl.BlockSpec((BN, D_OUT), lambda g: (g, 0)),
        out_shape=jax.ShapeDtypeStruct((N_NODES, D_OUT), jnp.float32),
    )(agg2p, agg2p, degp, degp, hr)


def kernel(x, edge_index, W1_l, W1_r, b1, W2_l, W2_r, b2):
    s2, d2, s2a, s2b = _tc_edge_prep(edge_index.astype(jnp.int32))

    w2l = jnp.pad(W2_l, ((0, 0), (0, D_OUT_PAD - D_OUT)))
    w2r = jnp.pad(W2_r, ((0, 0), (0, D_OUT_PAD - D_OUT)))
    b1r = b1.reshape(1, D_HID)
    b2r = jnp.pad(b2, (0, D_OUT_PAD - D_OUT)).reshape(1, D_OUT_PAD)

    HW = D_IN // 2
    zx = jnp.zeros((RPT, HW), jnp.float32)
    zd = jnp.zeros((RPT, 16), jnp.float32)
    z2 = jnp.zeros((RPT, D_OUT_PAD), jnp.float32)
    erow = jnp.zeros((EB, 16), jnp.float32).at[:, 0].set(1.0)

    x2 = x.reshape(2 * N_NODES, HW)
    aggp, degflat = _sc_aggregate(
        [x2, x2], [s2a, s2b], d2, zx, HW, True, 5, zdeg=zd, erow=erow)
    degp = degflat.reshape(NC * R // 8, 128)

    p2, hr = _tc_layer1(aggp, degp, x, W1_l, W1_r, b1r, w2l, w2r, b2r)

    (agg2p,) = _sc_aggregate([p2], [s2], d2, z2, D_OUT_PAD, False, 8)

    return _tc_layer2(agg2p, degp, hr)


# speedup vs baseline: 19.5380x; 1.0233x over previous
import jax
import jax.numpy as jnp
from jax import lax
from jax.experimental import pallas as pl
from jax.experimental.pallas import tpu as pltpu
from jax.experimental.pallas import tpu_sc as plsc

N_NODES = 10000
N_EDGES = 320000
D_IN = 128
D_HID = 256
D_OUT = 40
D_OUT_PAD = 48

NC = 2
NS = 16
NW = NC * NS

R = 10240
RPT = R // NS
EB = 128
EP = 327680
K = EP // (NW * EB)
EROWS = EP // EB

BN = 1024
NG = (N_NODES + BN - 1) // BN


def _sc_aggregate(tables, srcs, dsti, zrows, width, with_deg, nbuf,
                  zdeg=None, erow=None):
    npass = len(tables)
    assert len(srcs) == npass
    mesh = plsc.VectorSubcoreMesh(
        core_axis_name="c", subcore_axis_name="s", num_cores=NC, num_subcores=NS
    )
    out_type = [jax.ShapeDtypeStruct((NC * R, 128), jnp.float32)]
    scratch = [pltpu.VMEM((K, EB), jnp.int32) for _ in range(npass)]
    scratch += [
        pltpu.VMEM((K, EB), jnp.int32),
        pltpu.VMEM((nbuf, EB, width), jnp.float32),
        pltpu.VMEM_SHARED((R, width), jnp.float32),
    ]
    scratch += [pltpu.SemaphoreType.DMA] * (2 * nbuf)
    if with_deg:
        out_type.append(jax.ShapeDtypeStruct((NC * R, 16), jnp.float32))
        scratch += [
            pltpu.VMEM((EB, 16), jnp.float32),
            pltpu.VMEM_SHARED((R, 16), jnp.float32),
        ]

    def body(*refs):
        table_h = refs[:npass]
        src_h = refs[npass:2 * npass]
        if with_deg:
            (dst_h, z_h, zd_h, erow_h, agg_o, deg_o, *refs2) = refs[2 * npass:]
            src_v = refs2[:npass]
            (dst_v, rows_v, acc, *rest) = refs2[npass:]
            gsem = rest[:nbuf]
            ssem = rest[nbuf:2 * nbuf]
            erow_v, dacc = rest[2 * nbuf:]
        else:
            (dst_h, z_h, agg_o, *refs2) = refs[2 * npass:]
            src_v = refs2[:npass]
            (dst_v, rows_v, acc, *rest) = refs2[npass:]
            gsem = rest[:nbuf]
            ssem = rest[nbuf:2 * nbuf]
        c = lax.axis_index("c")
        s = lax.axis_index("s")
        w = s * NC + c
        r0 = s * RPT
        for p in range(npass):
            pltpu.sync_copy(src_h[p].at[pl.ds(w * K, K)], src_v[p])
        pltpu.sync_copy(dst_h.at[pl.ds(w * K, K)], dst_v)
        if with_deg:
            pltpu.sync_copy(zd_h, dacc.at[pl.ds(r0, RPT)])
            pltpu.sync_copy(erow_h, erow_v)

        for p in range(npass):
            deg_pass = with_deg and p == 0
            pltpu.sync_copy(z_h, acc.at[pl.ds(r0, RPT)])
            plsc.subcore_barrier()

            def gather(j, b, p=p):
                return pltpu.async_copy(table_h[p].at[src_v[p].at[j]],
                                        rows_v.at[b], gsem[b])

            for b in range(nbuf):
                gather(b, b)

            def step(t, carry, p=p, deg_pass=deg_pass, gather=gather):
                j0 = t * nbuf
                scat = []
                for b in range(nbuf):
                    j = j0 + b
                    pltpu.make_async_copy(table_h[p].at[src_v[p].at[j]],
                                          rows_v.at[b], gsem[b]).wait()
                    scat.append(pltpu.async_copy(
                        rows_v.at[b], acc.at[dst_v.at[j]], ssem[b], add=True))
                    if deg_pass:
                        scat.append(pltpu.async_copy(
                            erow_v, dacc.at[dst_v.at[j]], ssem[b], add=True))
                for b in range(nbuf):
                    j = j0 + nbuf + b

                    @pl.when(j < K)
                    def _():
                        nb = 2 if deg_pass else 1
                        for d in scat[b * nb:(b + 1) * nb]:
                            d.wait()
                        gather(j, b)

                return carry

            lax.fori_loop(0, K // nbuf, step, 0)
            for b in range(nbuf):
                j = K - nbuf + b
                pltpu.make_async_copy(rows_v.at[b], acc.at[dst_v.at[j]],
                                      ssem[b]).wait()
                if deg_pass:
                    pltpu.make_async_copy(erow_v, dacc.at[dst_v.at[j]],
                                          ssem[b]).wait()
            plsc.subcore_barrier()
            pltpu.sync_copy(
                acc.at[pl.ds(r0, RPT)],
                agg_o.at[pl.ds(c * R + r0, RPT), pl.ds(p * width, width)])
        if with_deg:
            pltpu.sync_copy(dacc.at[pl.ds(r0, RPT)],
                            deg_o.at[pl.ds(c * R + r0, RPT)])

    args = list(tables) + list(srcs) + [dsti, zrows]
    if with_deg:
        args += [zdeg, erow]
    run = pl.kernel(
        body,
        out_type=out_type,
        mesh=mesh,
        scratch_types=scratch,
        compiler_params=pltpu.CompilerParams(use_tc_tiling_on_sc=False),
    )
    return run(*args)


def _tc_edge_prep(edge_index):
    ER = N_EDGES // EB

    def body(e_r, s_o, d_o, sa_o, sb_o):
        e = e_r[...]
        e0 = e[0].reshape(ER, EB)
        e1 = e[1].reshape(ER, EB)
        r = (lax.broadcasted_iota(jnp.int32, (EROWS - ER, EB), 0) * EB
             + lax.broadcasted_iota(jnp.int32, (EROWS - ER, EB), 1))
        s = jnp.concatenate([e0, r % N_NODES], axis=0)
        s_o[...] = s
        d_o[...] = jnp.concatenate([e1, N_NODES + r % (R - N_NODES)], axis=0)
        sa_o[...] = 2 * s
        sb_o[...] = 2 * s + 1

    return pl.pallas_call(
        body,
        grid=(1,),
        in_specs=[pl.BlockSpec((2, N_EDGES), lambda g: (0, 0))],
        out_specs=[pl.BlockSpec((EROWS, EB), lambda g: (0, 0))] * 4,
        out_shape=[jax.ShapeDtypeStruct((EROWS, EB), jnp.int32)] * 4,
    )(edge_index)


def _apply_rdeg(a, d0, d1):
    d = (d0 + d1).reshape(BN // 8, 8, 16)
    rdeg = 1.0 / jnp.maximum(d[:, :, 0:1], 1.0)
    w = a.shape[-1]
    return (a.reshape(BN // 8, 8, w) * rdeg).reshape(BN, w)


def _tc_layer1(aggp, degp, x, w1l, w1r, b1, w2l, w2r, b2):
    def body(a0_r, a1_r, d0_r, d1_r, x_r, wl_r, wr_r, b1_r, w2l_r, w2r_r,
             b2_r, p2_o, hr_o):
        m = _apply_rdeg(a0_r[...] + a1_r[...], d0_r[...], d1_r[...])
        h = jnp.dot(m, wl_r[...], preferred_element_type=jnp.float32)
        h += jnp.dot(x_r[...], wr_r[...], preferred_element_type=jnp.float32)
        h = jnp.maximum(h + b1_r[...], 0.0)
        p2_o[...] = jnp.dot(h, w2l_r[...], preferred_element_type=jnp.float32)
        hr_o[...] = jnp.dot(h, w2r_r[...],
                            preferred_element_type=jnp.float32) + b2_r[...]

    GB = R // BN
    DB = BN // 8
    return pl.pallas_call(
        body,
        grid=(NG,),
        in_specs=[
            pl.BlockSpec((BN, D_IN), lambda g: (g, 0)),
            pl.BlockSpec((BN, D_IN), lambda g: (GB + g, 0)),
            pl.BlockSpec((DB, 128), lambda g: (g, 0)),
            pl.BlockSpec((DB, 128), lambda g: (GB + g, 0)),
            pl.BlockSpec((BN, D_IN), lambda g: (g, 0)),
            pl.BlockSpec((D_IN, D_HID), lambda g: (0, 0)),
            pl.BlockSpec((D_IN, D_HID), lambda g: (0, 0)),
            pl.BlockSpec((1, D_HID), lambda g: (0, 0)),
            pl.BlockSpec((D_HID, D_OUT_PAD), lambda g: (0, 0)),
            pl.BlockSpec((D_HID, D_OUT_PAD), lambda g: (0, 0)),
            pl.BlockSpec((1, D_OUT_PAD), lambda g: (0, 0)),
        ],
        out_specs=[
            pl.BlockSpec((BN, D_OUT_PAD), lambda g: (g, 0)),
            pl.BlockSpec((BN, D_OUT_PAD), lambda g: (g, 0)),
        ],
        out_shape=[
            jax.ShapeDtypeStruct((N_NODES, D_OUT_PAD), jnp.float32),
            jax.ShapeDtypeStruct((N_NODES, D_OUT_PAD), jnp.float32),
        ],
    )(aggp, aggp, degp, degp, x, w1l, w1r, b1, w2l, w2r, b2)


def _tc_layer2(agg2p, degp, hr):
    def body(a0_r, a1_r, d0_r, d1_r, hr_r, out_o):
        a = a0_r[...][:, 0:D_OUT_PAD] + a1_r[...][:, 0:D_OUT_PAD]
        z = _apply_rdeg(a, d0_r[...], d1_r[...]) + hr_r[...]
        col = lax.broadcasted_iota(jnp.int32, (BN, D_OUT_PAD), 1)
        z = jnp.where(col < D_OUT, z, -1e30)
        m = jnp.max(z, axis=-1, keepdims=True)
        e = jnp.exp(z - m)
        lse = jnp.log(jnp.sum(e, axis=-1, keepdims=True))
        out_o[...] = (z - m - lse)[:, 0:D_OUT].T

    GB = R // BN
    DB = BN // 8
    return pl.pallas_call(
        body,
        grid=(NG,),
        in_specs=[
            pl.BlockSpec((BN, 128), lambda g: (g, 0)),
            pl.BlockSpec((BN, 128), lambda g: (GB + g, 0)),
            pl.BlockSpec((DB, 128), lambda g: (g, 0)),
            pl.BlockSpec((DB, 128), lambda g: (GB + g, 0)),
            pl.BlockSpec((BN, D_OUT_PAD), lambda g: (g, 0)),
        ],
        out_specs=pl.BlockSpec((D_OUT, BN), lambda g: (0, g)),
        out_shape=jax.ShapeDtypeStruct((D_OUT, N_NODES), jnp.float32),
    )(agg2p, agg2p, degp, degp, hr)


def kernel(x, edge_index, W1_l, W1_r, b1, W2_l, W2_r, b2):
    s2, d2, s2a, s2b = _tc_edge_prep(edge_index.astype(jnp.int32))

    w2l = jnp.pad(W2_l, ((0, 0), (0, D_OUT_PAD - D_OUT)))
    w2r = jnp.pad(W2_r, ((0, 0), (0, D_OUT_PAD - D_OUT)))
    b1r = b1.reshape(1, D_HID)
    b2r = jnp.pad(b2, (0, D_OUT_PAD - D_OUT)).reshape(1, D_OUT_PAD)

    HW = D_IN // 2
    zx = jnp.zeros((RPT, HW), jnp.float32)
    zd = jnp.zeros((RPT, 16), jnp.float32)
    z2 = jnp.zeros((RPT, D_OUT_PAD), jnp.float32)
    erow = jnp.zeros((EB, 16), jnp.float32).at[:, 0].set(1.0)

    x2 = x.reshape(2 * N_NODES, HW)
    aggp, degflat = _sc_aggregate(
        [x2, x2], [s2a, s2b], d2, zx, HW, True, 5, zdeg=zd, erow=erow)
    degp = degflat.reshape(NC * R // 8, 128)

    p2, hr = _tc_layer1(aggp, degp, x, W1_l, W1_r, b1r, w2l, w2r, b2r)

    (agg2p,) = _sc_aggregate([p2], [s2], d2, z2, D_OUT_PAD, False, 10)

    return _tc_layer2(agg2p, degp, hr).T
